# Initial kernel scaffold; baseline (speedup 1.0000x reference)
import jax
import jax.numpy as jnp
from jax.experimental import pallas as pl


def kernel(x, edge_index, edge_attr, W1, b1, g1, be1, W2, b2, g2, be2, Wf, bf, Wo, bo):
    raise NotImplementedError("write your pallas kernel here")


# speedup vs baseline: 3.2877x; 3.2877x over previous
import functools

import jax
import jax.numpy as jnp
from jax import lax
from jax.experimental import pallas as pl
from jax.experimental.pallas import tpu as pltpu
from jax.experimental.pallas import tpu_sc as plsc

N = 10000
NP = 10008
E = 320000
EP = 16 * 164 * 128
F_IN = 128
H = 1024
FC = 512
OUT = 10

NC = 2
NS = 16
ECH = 128
NBS = EP // (2 * NS * ECH)
ROWS_PER_SUB = 624
TAIL_OFF = NS * ROWS_PER_SUB
TAIL = NP - TAIL_OFF
R = 1112
GB = NP // R
EPS = 1e-5

_MESH = plsc.VectorSubcoreMesh(core_axis_name="c", subcore_axis_name="s")


def _init_acc(init_view, acc_sh, sid):
    pltpu.sync_copy(init_view.at[pl.ds(sid * ROWS_PER_SUB, ROWS_PER_SUB)],
                    acc_sh.at[pl.ds(sid * ROWS_PER_SUB, ROWS_PER_SUB)])

    @pl.when(sid == NS - 1)
    def _():
        pltpu.sync_copy(init_view.at[pl.ds(TAIL_OFF, TAIL)],
                        acc_sh.at[pl.ds(TAIL_OFF, TAIL)])


def _writeback(acc_sh, out_view, sid):
    pltpu.sync_copy(acc_sh.at[pl.ds(sid * ROWS_PER_SUB, ROWS_PER_SUB)],
                    out_view.at[pl.ds(sid * ROWS_PER_SUB, ROWS_PER_SUB)])

    @pl.when(sid == NS - 1)
    def _():
        pltpu.sync_copy(acc_sh.at[pl.ds(TAIL_OFF, TAIL)],
                        out_view.at[pl.ds(TAIL_OFF, TAIL)])


def _unpack_indices(pidx, out, off, nrows, shift):
    def row(r, carry):
        for k in range(ECH // 16):
            v = pidx[off + r, pl.ds(k * 16, 16)]
            if shift:
                v = lax.shift_right_logical(v, shift)
            out[r, pl.ds(k * 16, 16)] = v & 0x3FFF
        return carry

    lax.fori_loop(0, nrows, row, 0)


@functools.partial(
    pl.kernel,
    out_type=jax.ShapeDtypeStruct((NC, NP, 128), jnp.float32),
    mesh=_MESH,
    scratch_types=[
        pltpu.VMEM((NBS, ECH), jnp.int32),
        pltpu.VMEM((NBS // 2, ECH), jnp.int32),
        pltpu.VMEM((NBS // 2, ECH), jnp.int32),
        pltpu.VMEM((ECH, 128), jnp.float32),
        pltpu.VMEM_SHARED((NP, 128), jnp.float32),
        pltpu.SemaphoreType.DMA,
    ],
)
def _agg_kernel(xs_hbm, ep_hbm, init_hbm, agg_hbm,
                pidx, sidx, didx, rows0, acc_sh, sem0):
    cid = lax.axis_index("c")
    sid = lax.axis_index("s")
    half = NBS // 2
    pltpu.sync_copy(ep_hbm.at[sid], pidx)
    _unpack_indices(pidx, sidx, cid * half, half, 14)
    _unpack_indices(pidx, didx, cid * half, half, 0)
    _init_acc(init_hbm.at[cid], acc_sh, sid)
    plsc.subcore_barrier()

    def body(j, carry):
        pltpu.async_copy(xs_hbm.at[sidx.at[j]], rows0, sem0).wait()
        pltpu.sync_copy(rows0, acc_sh.at[didx.at[j]], add=True)
        return carry

    lax.fori_loop(0, half, body, 0)
    plsc.subcore_barrier()
    _writeback(acc_sh, agg_hbm.at[cid], sid)


_NCH2 = H // 128


def _dinv_of(degp_blk):
    return lax.rsqrt(degp_blk[0, :, 0] + degp_blk[1, :, 0])


def _row_mask(r):
    i = pl.program_id(0)
    rows = i * R + lax.broadcasted_iota(jnp.int32, (r, 1), 0)
    return rows < N


def _t0_body(x_ref, degp_ref, xs_ref):
    dinv = _dinv_of(degp_ref[...])
    xs_ref[...] = x_ref[...] * dinv[:, None]


def _t1_body(agg_ref, degp_ref, w_ref, b_ref, h_ref, s_ref, q_ref):
    dinv = _dinv_of(degp_ref[...])
    a = agg_ref[0] + agg_ref[1]
    _mm_stats(a, dinv, w_ref, b_ref, h_ref, s_ref, q_ref,
              lax.Precision.HIGHEST)


def _t3_body(*refs):
    aggs = refs[:_NCH2]
    degp_ref, w_ref, b_ref, h_ref, s_ref, q_ref = refs[_NCH2:]
    dinv = _dinv_of(degp_ref[...])
    a = jnp.concatenate(
        [aggs[k][0] + aggs[k][1] for k in range(_NCH2)], axis=1)
    _mm_stats(a, dinv, w_ref, b_ref, h_ref, s_ref, q_ref,
              lax.Precision.HIGHEST)


def _mm_stats(a, dinv, w_ref, b_ref, h_ref, s_ref, q_ref,
              precision=None):
    pre = a * dinv[:, None]
    h = jnp.dot(pre, w_ref[...], preferred_element_type=jnp.float32,
                precision=precision) + b_ref[...]
    h = jnp.where(_row_mask(h.shape[0]), h, 0.0)
    h_ref[...] = h

    @pl.when(pl.program_id(0) == 0)
    def _():
        s_ref[...] = jnp.zeros_like(s_ref)
        q_ref[...] = jnp.zeros_like(q_ref)

    s_ref[...] += jnp.sum(h, axis=0, keepdims=True)
    q_ref[...] += jnp.sum(h * h, axis=0, keepdims=True)


def _bn_relu(h, s, q, g, be):
    m = s * (1.0 / N)
    v = q * (1.0 / N) - m * m
    inv = lax.rsqrt(v + EPS)
    return jax.nn.relu((h - m) * inv * g + be)


def _t2_body(h_ref, s_ref, q_ref, g_ref, be_ref, degp_ref, *xs_refs):
    h1 = _bn_relu(h_ref[...], s_ref[...], q_ref[...], g_ref[...], be_ref[...])
    dinv = _dinv_of(degp_ref[...])
    xs = jnp.where(_row_mask(h1.shape[0]), h1 * dinv[:, None], 0.0)
    for k in range(_NCH2):
        xs_refs[k][...] = xs[:, k * 128:(k + 1) * 128]


def _t4_body(h_ref, s_ref, q_ref, g_ref, be_ref, wf_ref, bf_ref,
             wo_ref, bo_ref, out_ref):
    h2 = _bn_relu(h_ref[...], s_ref[...], q_ref[...], g_ref[...], be_ref[...])
    h3 = jax.nn.relu(
        jnp.dot(h2, wf_ref[...], preferred_element_type=jnp.float32,
                precision=lax.Precision.HIGHEST) + bf_ref[...])
    out_ref[...] = (
        jnp.dot(h3, wo_ref[...], preferred_element_type=jnp.float32,
                precision=lax.Precision.HIGHEST) + bo_ref[...])


def _full2d(shape):
    return pl.BlockSpec(shape, lambda i: (0, 0))


def _rows2d(cols):
    return pl.BlockSpec((R, cols), lambda i: (i, 0))


_DEGP_SPEC = pl.BlockSpec((2, R, 128), lambda i: (0, i, 0))


def _t0(x_p, degp):
    return pl.pallas_call(
        _t0_body,
        grid=(GB,),
        in_specs=[_rows2d(F_IN), _DEGP_SPEC],
        out_specs=_rows2d(F_IN),
        out_shape=jax.ShapeDtypeStruct((NP, F_IN), jnp.float32),
    )(x_p, degp)


def _t1(agg, degp, w, b):
    return pl.pallas_call(
        _t1_body,
        grid=(GB,),
        in_specs=[_DEGP_SPEC, _DEGP_SPEC,
                  _full2d((F_IN, H)), _full2d((1, H))],
        out_specs=[_rows2d(H), _full2d((1, H)), _full2d((1, H))],
        out_shape=[jax.ShapeDtypeStruct((NP, H), jnp.float32),
                   jax.ShapeDtypeStruct((1, H), jnp.float32),
                   jax.ShapeDtypeStruct((1, H), jnp.float32)],
    )(agg, degp, w, b)


def _t3(agg2s, degp, w, b):
    return pl.pallas_call(
        _t3_body,
        grid=(GB,),
        in_specs=([_DEGP_SPEC] * _NCH2
                  + [_DEGP_SPEC, _full2d((H, H)), _full2d((1, H))]),
        out_specs=[_rows2d(H), _full2d((1, H)), _full2d((1, H))],
        out_shape=[jax.ShapeDtypeStruct((NP, H), jnp.float32),
                   jax.ShapeDtypeStruct((1, H), jnp.float32),
                   jax.ShapeDtypeStruct((1, H), jnp.float32)],
    )(*agg2s, degp, w, b)


def _t2(hpre, s, q, g, be, degp):
    return pl.pallas_call(
        _t2_body,
        grid=(GB,),
        in_specs=[_rows2d(H), _full2d((1, H)), _full2d((1, H)),
                  _full2d((1, H)), _full2d((1, H)), _DEGP_SPEC],
        out_specs=[_rows2d(128)] * _NCH2,
        out_shape=[jax.ShapeDtypeStruct((NP, 128), jnp.float32)] * _NCH2,
    )(hpre, s, q, g, be, degp)


def _t4(hpre, s, q, g, be, wf, bf, wo, bo):
    return pl.pallas_call(
        _t4_body,
        grid=(GB,),
        in_specs=[_rows2d(H), _full2d((1, H)), _full2d((1, H)),
                  _full2d((1, H)), _full2d((1, H)),
                  _full2d((H, FC)), _full2d((1, FC)),
                  _full2d((FC, OUT)), _full2d((1, OUT))],
        out_specs=pl.BlockSpec((R, OUT), lambda i: (i, 0)),
        out_shape=jax.ShapeDtypeStruct((N, OUT), jnp.float32),
    )(hpre, s, q, g, be, wf, bf, wo, bo)


def kernel(x, edge_index, edge_attr, W1, b1, g1, be1, W2, b2, g2, be2,
           Wf, bf, Wo, bo):
    del edge_attr
    i32 = jnp.int32
    loop = jnp.arange(N, dtype=i32)
    pad = jnp.full((EP - E - N,), N, dtype=i32)
    src = jnp.concatenate([edge_index[0].astype(i32), loop, pad])
    dst = jnp.concatenate([edge_index[1].astype(i32), loop, pad])
    epp = ((src << 14) | dst).reshape(2, NS, NBS, ECH)
    ep_a, ep_b = epp[0], epp[1]
    x_p = jnp.pad(x, ((0, NP - N), (0, 0)))
    z2 = jnp.zeros((NC, NP, 128), jnp.float32)
    ones_np = jnp.ones((NP, 128), jnp.float32)

    def agg(xs):
        return _agg_kernel(xs, ep_b, _agg_kernel(xs, ep_a, z2))

    degp = agg(ones_np)
    xs1 = _t0(x_p, degp)
    agg1 = agg(xs1)
    hpre1, s1, q1 = _t1(agg1, degp, W1, b1.reshape(1, H))
    xs2s = _t2(hpre1, s1, q1, g1.reshape(1, H), be1.reshape(1, H), degp)
    agg2s = [agg(xs2s[k]) for k in range(_NCH2)]
    hpre2, s2, q2 = _t3(agg2s, degp, W2, b2.reshape(1, H))
    return _t4(hpre2, s2, q2, g2.reshape(1, H), be2.reshape(1, H),
               Wf, bf.reshape(1, FC), Wo, bo.reshape(1, OUT))
